# Initial kernel scaffold; baseline (speedup 1.0000x reference)
import jax
import jax.numpy as jnp
from jax.experimental import pallas as pl


def kernel(x, gate_w, gate_b, w1, b1, w2, b2):
    raise NotImplementedError("write your pallas kernel here")


# speedup vs baseline: 6.4332x; 6.4332x over previous
import functools

import jax
import jax.numpy as jnp
from jax.experimental import pallas as pl
from jax.experimental.pallas import tpu as pltpu

E = 64
K = 2
BM = 256


def _router_body(x_ref, gwt_ref, gb_ref, idx_ref, gate_ref):
    logits = (
        jnp.dot(x_ref[...], gwt_ref[...], preferred_element_type=jnp.float32)
        + gb_ref[0, :][None, :]
    )
    tb = logits.shape[0]
    cols = jax.lax.broadcasted_iota(jnp.int32, logits.shape, 1)
    m1 = jnp.max(logits, axis=1)
    a1 = jnp.argmax(logits, axis=1).astype(jnp.int32)
    masked = jnp.where(cols == a1[:, None], -jnp.inf, logits)
    m2 = jnp.max(masked, axis=1)
    a2 = jnp.argmax(masked, axis=1).astype(jnp.int32)
    g1 = 1.0 / (1.0 + jnp.exp(m2 - m1))
    g2 = 1.0 - g1
    two = jax.lax.broadcasted_iota(jnp.int32, (tb, 2), 1)
    idx_ref[...] = jnp.where(two == 0, a1[:, None], a2[:, None])
    gate_ref[...] = jnp.where(two == 0, g1[:, None], g2[:, None])


def _router(xf, gwt, gb2):
    T, D = xf.shape
    TB = 512
    grid = (T // TB,)
    return pl.pallas_call(
        _router_body,
        grid=grid,
        in_specs=[
            pl.BlockSpec((TB, D), lambda i: (i, 0)),
            pl.BlockSpec((D, E), lambda i: (0, 0)),
            pl.BlockSpec((1, E), lambda i: (0, 0)),
        ],
        out_specs=[
            pl.BlockSpec((TB, 2), lambda i: (i, 0)),
            pl.BlockSpec((TB, 2), lambda i: (i, 0)),
        ],
        out_shape=[
            jax.ShapeDtypeStruct((T, 2), jnp.int32),
            jax.ShapeDtypeStruct((T, 2), jnp.float32),
        ],
    )(xf, gwt, gb2)


def _ffn_body(be_ref, nbt_ref, xg_ref, w1_ref, b1_ref, w2_ref, b2_ref, wgt_ref, y_ref):
    b = pl.program_id(0)

    @pl.when(b < nbt_ref[0])
    def _():
        xb = xg_ref[...]
        h = jnp.dot(xb, w1_ref[...], preferred_element_type=jnp.float32)
        h = jnp.maximum(h + b1_ref[0, :][None, :], 0.0)
        y = jnp.dot(h, w2_ref[...], preferred_element_type=jnp.float32)
        y = y + b2_ref[0, :][None, :]
        y_ref[...] = y * wgt_ref[0, 0, :][:, None]


def _ffn(be, nbt, xg, w1, b1r, w2, b2r, wgt3):
    PAD_T, D = xg.shape
    NBLK = PAD_T // BM
    F = w1.shape[2]
    grid_spec = pltpu.PrefetchScalarGridSpec(
        num_scalar_prefetch=2,
        grid=(NBLK,),
        in_specs=[
            pl.BlockSpec((BM, D), lambda b, be, nbt: (b, 0)),
            pl.BlockSpec((None, D, F), lambda b, be, nbt: (be[b], 0, 0)),
            pl.BlockSpec((None, 1, F), lambda b, be, nbt: (be[b], 0, 0)),
            pl.BlockSpec((None, F, D), lambda b, be, nbt: (be[b], 0, 0)),
            pl.BlockSpec((None, 1, D), lambda b, be, nbt: (be[b], 0, 0)),
            pl.BlockSpec((1, 1, BM), lambda b, be, nbt: (b, 0, 0)),
        ],
        out_specs=pl.BlockSpec((BM, D), lambda b, be, nbt: (b, 0)),
    )
    return pl.pallas_call(
        _ffn_body,
        grid_spec=grid_spec,
        out_shape=jax.ShapeDtypeStruct((PAD_T, D), jnp.float32),
        compiler_params=pltpu.CompilerParams(
            dimension_semantics=("arbitrary",),
        ),
    )(be, nbt, xg, w1, b1r, w2, b2r, wgt3)


def kernel(x, gate_w, gate_b, w1, b1, w2, b2):
    Bb, S, D = x.shape
    T = Bb * S
    A = T * K
    NBLK = A // BM + E
    PAD_T = NBLK * BM
    xf = x.reshape(T, D)

    top_idx, gates = _router(xf, gate_w.T, gate_b.reshape(1, E))

    e_flat = top_idx.reshape(-1)
    order = jnp.argsort(e_flat, stable=True).astype(jnp.int32)
    counts = jnp.bincount(e_flat, length=E).astype(jnp.int32)
    offs = jnp.cumsum(counts) - counts
    nb = (counts + BM - 1) // BM
    cnb = jnp.cumsum(nb)
    nbt = cnb[-1]
    nb_off = cnb - nb
    bidx = jnp.arange(NBLK, dtype=jnp.int32)
    be = jnp.searchsorted(cnb, bidx, side="right").astype(jnp.int32)
    last_e = jnp.searchsorted(cnb, nbt - 1, side="right").astype(jnp.int32)
    be = jnp.where(bidx < nbt, be, last_e)

    p = jnp.arange(PAD_T, dtype=jnp.int32)
    ep = be[p // BM]
    j = p - nb_off[ep] * BM
    valid = (p // BM < nbt) & (j < counts[ep])
    src = jnp.clip(offs[ep] + j, 0, A - 1)
    a = order[src]
    tok = jnp.where(valid, a // K, 0)
    wgt = jnp.where(valid, gates.reshape(-1)[a], 0.0)
    slot_of = (
        jnp.zeros((A,), jnp.int32)
        .at[jnp.where(valid, a, A)]
        .set(p, mode="drop")
        .reshape(T, K)
    )

    xg = xf[tok]

    yg = _ffn(
        be,
        nbt.reshape(1),
        xg,
        w1,
        b1.reshape(E, 1, -1),
        w2,
        b2.reshape(E, 1, -1),
        wgt.reshape(NBLK, 1, BM),
    )

    out = yg[slot_of[:, 0]] + yg[slot_of[:, 1]]
    return out.reshape(Bb, S, D)


# speedup vs baseline: 6.4602x; 1.0042x over previous
import functools

import jax
import jax.numpy as jnp
from jax.experimental import pallas as pl
from jax.experimental.pallas import tpu as pltpu

E = 64
K = 2
BM = 256


def _router_body(x_ref, gwt_ref, gb_ref, idx_ref, gate_ref):
    logits = (
        jnp.dot(x_ref[...], gwt_ref[...], preferred_element_type=jnp.float32)
        + gb_ref[0, :][None, :]
    )
    tb = logits.shape[0]
    cols = jax.lax.broadcasted_iota(jnp.int32, logits.shape, 1)
    m1 = jnp.max(logits, axis=1)
    a1 = jnp.argmax(logits, axis=1).astype(jnp.int32)
    masked = jnp.where(cols == a1[:, None], -jnp.inf, logits)
    m2 = jnp.max(masked, axis=1)
    a2 = jnp.argmax(masked, axis=1).astype(jnp.int32)
    g1 = 1.0 / (1.0 + jnp.exp(m2 - m1))
    g2 = 1.0 - g1
    two = jax.lax.broadcasted_iota(jnp.int32, (tb, 2), 1)
    idx_ref[...] = jnp.where(two == 0, a1[:, None], a2[:, None])
    gate_ref[...] = jnp.where(two == 0, g1[:, None], g2[:, None])


def _router(xf, gwt, gb2):
    T, D = xf.shape
    TB = 512
    grid = (T // TB,)
    return pl.pallas_call(
        _router_body,
        grid=grid,
        in_specs=[
            pl.BlockSpec((TB, D), lambda i: (i, 0)),
            pl.BlockSpec((D, E), lambda i: (0, 0)),
            pl.BlockSpec((1, E), lambda i: (0, 0)),
        ],
        out_specs=[
            pl.BlockSpec((TB, 2), lambda i: (i, 0)),
            pl.BlockSpec((TB, 2), lambda i: (i, 0)),
        ],
        out_shape=[
            jax.ShapeDtypeStruct((T, 2), jnp.int32),
            jax.ShapeDtypeStruct((T, 2), jnp.float32),
        ],
    )(xf, gwt, gb2)


def _ffn_body(be_ref, nbt_ref, xg_ref, w1_ref, b1_ref, w2_ref, b2_ref, wgt_ref, y_ref):
    b = pl.program_id(0)

    @pl.when(b < nbt_ref[0])
    def _():
        xb = xg_ref[...].astype(jnp.bfloat16)
        h = jnp.dot(
            xb, w1_ref[...].astype(jnp.bfloat16), preferred_element_type=jnp.float32
        )
        h = jnp.maximum(h + b1_ref[0, :][None, :], 0.0)
        y = jnp.dot(
            h.astype(jnp.bfloat16),
            w2_ref[...].astype(jnp.bfloat16),
            preferred_element_type=jnp.float32,
        )
        y = y + b2_ref[0, :][None, :]
        y_ref[...] = y * wgt_ref[0, 0, :][:, None]


def _ffn(be, nbt, xg, w1, b1r, w2, b2r, wgt3):
    PAD_T, D = xg.shape
    NBLK = PAD_T // BM
    F = w1.shape[2]
    grid_spec = pltpu.PrefetchScalarGridSpec(
        num_scalar_prefetch=2,
        grid=(NBLK,),
        in_specs=[
            pl.BlockSpec((BM, D), lambda b, be, nbt: (b, 0)),
            pl.BlockSpec((None, D, F), lambda b, be, nbt: (be[b], 0, 0)),
            pl.BlockSpec((None, 1, F), lambda b, be, nbt: (be[b], 0, 0)),
            pl.BlockSpec((None, F, D), lambda b, be, nbt: (be[b], 0, 0)),
            pl.BlockSpec((None, 1, D), lambda b, be, nbt: (be[b], 0, 0)),
            pl.BlockSpec((1, 1, BM), lambda b, be, nbt: (b, 0, 0)),
        ],
        out_specs=pl.BlockSpec((BM, D), lambda b, be, nbt: (b, 0)),
    )
    return pl.pallas_call(
        _ffn_body,
        grid_spec=grid_spec,
        out_shape=jax.ShapeDtypeStruct((PAD_T, D), jnp.float32),
        compiler_params=pltpu.CompilerParams(
            dimension_semantics=("arbitrary",),
        ),
    )(be, nbt, xg, w1, b1r, w2, b2r, wgt3)


def kernel(x, gate_w, gate_b, w1, b1, w2, b2):
    Bb, S, D = x.shape
    T = Bb * S
    A = T * K
    NBLK = A // BM + E
    PAD_T = NBLK * BM
    xf = x.reshape(T, D)

    top_idx, gates = _router(xf, gate_w.T, gate_b.reshape(1, E))

    e_flat = top_idx.reshape(-1)
    order = jnp.argsort(e_flat, stable=True).astype(jnp.int32)
    counts = jnp.bincount(e_flat, length=E).astype(jnp.int32)
    offs = jnp.cumsum(counts) - counts
    nb = (counts + BM - 1) // BM
    cnb = jnp.cumsum(nb)
    nbt = cnb[-1]
    nb_off = cnb - nb
    bidx = jnp.arange(NBLK, dtype=jnp.int32)
    be = jnp.searchsorted(cnb, bidx, side="right").astype(jnp.int32)
    last_e = jnp.searchsorted(cnb, nbt - 1, side="right").astype(jnp.int32)
    be = jnp.where(bidx < nbt, be, last_e)

    p = jnp.arange(PAD_T, dtype=jnp.int32)
    ep = be[p // BM]
    j = p - nb_off[ep] * BM
    valid = (p // BM < nbt) & (j < counts[ep])
    src = jnp.clip(offs[ep] + j, 0, A - 1)
    a = order[src]
    tok = jnp.where(valid, a // K, 0)
    wgt = jnp.where(valid, gates.reshape(-1)[a], 0.0)
    slot_of = (
        jnp.zeros((A,), jnp.int32)
        .at[jnp.where(valid, a, A)]
        .set(p, mode="drop")
        .reshape(T, K)
    )

    xg = xf[tok]

    yg = _ffn(
        be,
        nbt.reshape(1),
        xg,
        w1,
        b1.reshape(E, 1, -1),
        w2,
        b2.reshape(E, 1, -1),
        wgt.reshape(NBLK, 1, BM),
    )

    out = yg[slot_of[:, 0]] + yg[slot_of[:, 1]]
    return out.reshape(Bb, S, D)


# speedup vs baseline: 12.5601x; 1.9442x over previous
import functools

import jax
import jax.numpy as jnp
from jax import lax
from jax.experimental import pallas as pl
from jax.experimental.pallas import tpu as pltpu
from jax.experimental.pallas import tpu_sc as plsc

E = 64
K = 2
BM = 128
NC = 2
NS = 16
NW = NC * NS


def _router_body(x_ref, gwt_ref, gb_ref, e0_ref, e1_ref, g0_ref, g1_ref):
    TB = x_ref.shape[0]
    NB = TB // 128
    f32 = jnp.float32

    logits = (
        jnp.dot(x_ref[...], gwt_ref[...], preferred_element_type=f32)
        + gb_ref[0, :][None, :]
    )
    logits3 = logits.reshape(NB, 128, E)
    cols3 = lax.broadcasted_iota(jnp.int32, (NB, 128, E), 2)
    m1 = jnp.max(logits3, axis=2)
    a1 = jnp.argmax(logits3, axis=2).astype(jnp.int32)
    masked3 = jnp.where(cols3 == a1[:, :, None], -jnp.inf, logits3)
    m2 = jnp.max(masked3, axis=2)
    a2 = jnp.argmax(masked3, axis=2).astype(jnp.int32)
    gA = 1.0 / (1.0 + jnp.exp(m2 - m1))
    e0_ref[...] = a1
    e1_ref[...] = a2
    g0_ref[...] = gA
    g1_ref[...] = 1.0 - gA


def _router(xf, gwt, gb2):
    T, D = xf.shape
    TB = 1024
    NBb = TB // 128
    return pl.pallas_call(
        _router_body,
        grid=(T // TB,),
        in_specs=[
            pl.BlockSpec((TB, D), lambda i: (i, 0)),
            pl.BlockSpec((D, E), lambda i: (0, 0)),
            pl.BlockSpec((1, E), lambda i: (0, 0)),
        ],
        out_specs=[
            pl.BlockSpec((NBb, 128), lambda i: (i, 0)),
            pl.BlockSpec((NBb, 128), lambda i: (i, 0)),
            pl.BlockSpec((NBb, 128), lambda i: (i, 0)),
            pl.BlockSpec((NBb, 128), lambda i: (i, 0)),
        ],
        out_shape=[
            jax.ShapeDtypeStruct((T // 128, 128), jnp.int32),
            jax.ShapeDtypeStruct((T // 128, 128), jnp.int32),
            jax.ShapeDtypeStruct((T // 128, 128), jnp.float32),
            jax.ShapeDtypeStruct((T // 128, 128), jnp.float32),
        ],
    )(xf, gwt, gb2)


def _make_dispatch(T, D, PAD_T):
    TPW = T // NW
    CH = 32
    mesh = plsc.VectorSubcoreMesh(core_axis_name="c", subcore_axis_name="s")

    @functools.partial(
        pl.kernel,
        mesh=mesh,
        out_type=[
            jax.ShapeDtypeStruct((PAD_T, D), jnp.float32),
            jax.ShapeDtypeStruct((PAD_T,), jnp.float32),
        ],
        scratch_types=[
            pltpu.VMEM((CH, D), jnp.float32),
            pltpu.VMEM((CH,), jnp.int32),
            pltpu.VMEM((CH,), jnp.int32),
            pltpu.VMEM((CH,), jnp.float32),
            pltpu.VMEM((CH,), jnp.float32),
            pltpu.SemaphoreType.DMA,
            pltpu.SemaphoreType.DMA,
            pltpu.SemaphoreType.DMA,
            pltpu.SemaphoreType.DMA,
        ],
    )
    def dispatch(xf_hbm, d0_hbm, d1_hbm, g0_hbm, g1_hbm, xg_hbm, wgt_hbm,
                 xb, i0, i1, ga, gb, s0, s1, s2, s3):
        wid = lax.axis_index("s") * NC + lax.axis_index("c")
        base = wid * TPW
        for c in range(TPW // CH):
            off = base + c * CH
            pltpu.sync_copy(d0_hbm.at[pl.ds(off, CH)], i0)
            pltpu.sync_copy(d1_hbm.at[pl.ds(off, CH)], i1)
            pltpu.sync_copy(g0_hbm.at[pl.ds(off, CH)], ga)
            pltpu.sync_copy(g1_hbm.at[pl.ds(off, CH)], gb)
            pltpu.sync_copy(xf_hbm.at[pl.ds(off, CH)], xb)
            cp0 = pltpu.async_copy(xb, xg_hbm.at[i0], s0)
            cp1 = pltpu.async_copy(xb, xg_hbm.at[i1], s1)
            cp2 = pltpu.async_copy(ga, wgt_hbm.at[i0], s2)
            cp3 = pltpu.async_copy(gb, wgt_hbm.at[i1], s3)
            cp0.wait()
            cp1.wait()
            cp2.wait()
            cp3.wait()

    return dispatch


def _make_combine(T, D, PAD_T):
    TPW = T // NW
    CH = 16
    mesh = plsc.VectorSubcoreMesh(core_axis_name="c", subcore_axis_name="s")

    @functools.partial(
        pl.kernel,
        mesh=mesh,
        out_type=jax.ShapeDtypeStruct((T, D), jnp.float32),
        scratch_types=[
            pltpu.VMEM((CH, D), jnp.float32),
            pltpu.VMEM((CH, D), jnp.float32),
            pltpu.VMEM((CH,), jnp.int32),
            pltpu.VMEM((CH,), jnp.int32),
            pltpu.SemaphoreType.DMA,
            pltpu.SemaphoreType.DMA,
        ],
    )
    def combine(yg_hbm, d0_hbm, d1_hbm, out_hbm, av, bv, i0, i1, s0, s1):
        wid = lax.axis_index("s") * NC + lax.axis_index("c")
        base = wid * TPW
        for c in range(TPW // CH):
            off = base + c * CH
            pltpu.sync_copy(d0_hbm.at[pl.ds(off, CH)], i0)
            pltpu.sync_copy(d1_hbm.at[pl.ds(off, CH)], i1)
            cp0 = pltpu.async_copy(yg_hbm.at[i0], av, s0)
            cp1 = pltpu.async_copy(yg_hbm.at[i1], bv, s1)
            cp0.wait()
            cp1.wait()

            def row(i, carry):
                for k in range(D // 16):
                    sl = pl.ds(k * 16, 16)
                    av[i, sl] = av[i, sl] + bv[i, sl]
                return carry

            lax.fori_loop(0, CH, row, 0)
            pltpu.sync_copy(av, out_hbm.at[pl.ds(off, CH)])

    return combine


def _ffn_body(be_ref, nbt_ref, xg_ref, w1_ref, b1_ref, w2_ref, b2_ref, wgt_ref, y_ref):
    b = pl.program_id(0)

    @pl.when(b < nbt_ref[0])
    def _():
        xb = xg_ref[...]
        h = jnp.dot(xb, w1_ref[...], preferred_element_type=jnp.float32)
        h = jnp.maximum(h + b1_ref[0, :][None, :], 0.0)
        y = jnp.dot(h, w2_ref[...], preferred_element_type=jnp.float32)
        y = y + b2_ref[0, :][None, :]
        y_ref[...] = y * wgt_ref[0, 0, :][:, None]


def _ffn(be, nbt, xg, w1, b1r, w2, b2r, wgt3):
    PAD_T, D = xg.shape
    NBLK = PAD_T // BM
    F = w1.shape[2]
    grid_spec = pltpu.PrefetchScalarGridSpec(
        num_scalar_prefetch=2,
        grid=(NBLK,),
        in_specs=[
            pl.BlockSpec((BM, D), lambda b, be, nbt: (b, 0)),
            pl.BlockSpec((None, D, F), lambda b, be, nbt: (be[b], 0, 0)),
            pl.BlockSpec((None, 1, F), lambda b, be, nbt: (be[b], 0, 0)),
            pl.BlockSpec((None, F, D), lambda b, be, nbt: (be[b], 0, 0)),
            pl.BlockSpec((None, 1, D), lambda b, be, nbt: (be[b], 0, 0)),
            pl.BlockSpec((1, 1, BM), lambda b, be, nbt: (b, 0, 0)),
        ],
        out_specs=pl.BlockSpec((BM, D), lambda b, be, nbt: (b, 0)),
    )
    return pl.pallas_call(
        _ffn_body,
        grid_spec=grid_spec,
        out_shape=jax.ShapeDtypeStruct((PAD_T, D), jnp.float32),
        compiler_params=pltpu.CompilerParams(
            dimension_semantics=("arbitrary",),
        ),
    )(be, nbt, xg, w1, b1r, w2, b2r, wgt3)


def kernel(x, gate_w, gate_b, w1, b1, w2, b2):
    Bb, S, D = x.shape
    T = Bb * S
    A = T * K
    NBLK = A // BM + E
    PAD_T = NBLK * BM
    xf = x.reshape(T, D)

    e0m, e1m, g0m, g1m = _router(xf, gate_w.T, gate_b.reshape(1, E))

    e_flat = jnp.stack([e0m.reshape(T), e1m.reshape(T)], axis=1).reshape(A)
    order = jnp.argsort(e_flat, stable=True).astype(jnp.int32)
    counts = jnp.bincount(e_flat, length=E).astype(jnp.int32)
    cum = jnp.cumsum(counts)
    offs = cum - counts
    nb = (counts + BM - 1) // BM
    cnb = jnp.cumsum(nb)
    nbt = cnb[-1]
    pad0 = (cnb - nb) * BM
    inv = jnp.zeros((A,), jnp.int32).at[order].set(jnp.arange(A, dtype=jnp.int32))
    dest = pad0[e_flat] + inv - offs[e_flat]
    d2 = dest.reshape(T, K)
    bidx = jnp.arange(NBLK, dtype=jnp.int32)
    be = jnp.searchsorted(cnb, jnp.minimum(bidx, nbt - 1), side="right").astype(
        jnp.int32
    )

    xg, wgt = _make_dispatch(T, D, PAD_T)(
        xf, d2[:, 0], d2[:, 1], g0m.reshape(T), g1m.reshape(T)
    )
    yg = _ffn(
        be,
        nbt.reshape(1),
        xg,
        w1,
        b1.reshape(E, 1, -1),
        w2,
        b2.reshape(E, 1, -1),
        wgt.reshape(NBLK, 1, BM),
    )
    out = _make_combine(T, D, PAD_T)(yg, d2[:, 0], d2[:, 1])
    return out.reshape(Bb, S, D)
